# baseline (speedup 1.0000x reference)
import functools

import jax
import jax.numpy as jnp
from jax import lax
from jax.experimental import pallas as pl
from jax.experimental.pallas import tpu as pltpu
from jax.experimental.pallas import tpu_sc as plsc

BATCH = 16384
EMBED = 64
BLK = 1024
SLOTS = 56


def _scalar(v, l):
    return lax.squeeze(lax.slice(v, (l,), (l + 1,)), (0,))


def _extract_body(nblocks, rem, ids_ref, u_ref, i_ref, uo_ref, io_ref):
    m = pl.program_id(0)
    cols = jax.lax.broadcasted_iota(jnp.int32, (BLK, SLOTS), 0) + m * BLK
    uids = ids_ref[0, 0, :SLOTS][None, :]
    iids = ids_ref[0, 0, SLOTS:][None, :]
    uoh = (cols == uids).astype(jnp.float32)
    ioh = (cols == iids).astype(jnp.float32)
    dn = (((0,), (1,)), ((), ()))

    @pl.when(m != nblocks - 1)
    def _():
        uo_ref[...] = lax.dot_general(uoh, u_ref[...], dn,
                                      preferred_element_type=jnp.float32)
        io_ref[...] = lax.dot_general(ioh, i_ref[...], dn,
                                      preferred_element_type=jnp.float32)

    @pl.when(m == nblocks - 1)
    def _():
        lcol = jax.lax.broadcasted_iota(jnp.int32, (EMBED, BLK), 1)
        xu = jnp.where(lcol < rem, u_ref[...], 0.0)
        xi = jnp.where(lcol < rem, i_ref[...], 0.0)
        uo_ref[...] = lax.dot_general(uoh, xu, dn,
                                      preferred_element_type=jnp.float32)
        io_ref[...] = lax.dot_general(ioh, xi, dn,
                                      preferred_element_type=jnp.float32)


def _extract(tabTu, tabTi, blk_ids, nblocks, interpret=False):
    rem = tabTu.shape[1] - (nblocks - 1) * BLK
    return pl.pallas_call(
        functools.partial(_extract_body, nblocks, rem),
        grid=(nblocks,),
        in_specs=[
            pl.BlockSpec((1, 1, 2 * SLOTS), lambda m: (m, 0, 0)),
            pl.BlockSpec((EMBED, BLK), lambda m: (0, m)),
            pl.BlockSpec((EMBED, BLK), lambda m: (0, m)),
        ],
        out_specs=[
            pl.BlockSpec((SLOTS, EMBED), lambda m: (m, 0)),
            pl.BlockSpec((SLOTS, EMBED), lambda m: (m, 0)),
        ],
        out_shape=[
            jax.ShapeDtypeStruct((nblocks * SLOTS, EMBED), jnp.float32),
            jax.ShapeDtypeStruct((nblocks * SLOTS, EMBED), jnp.float32),
        ],
        compiler_params=pltpu.CompilerParams(
            dimension_semantics=("arbitrary",)),
        interpret=interpret,
    )(blk_ids, tabTu, tabTi)


@functools.lru_cache(maxsize=None)
def _make_gather(num_slots):
    info = plsc.get_sparse_core_info()
    nc, ns = info.num_cores, info.num_subcores
    nw = nc * ns
    bpw = BATCH // nw

    mesh = plsc.VectorSubcoreMesh(core_axis_name="c", subcore_axis_name="s")

    @functools.partial(
        pl.kernel,
        mesh=mesh,
        out_type=[
            jax.ShapeDtypeStruct((BATCH, EMBED), jnp.float32),
            jax.ShapeDtypeStruct((BATCH, EMBED), jnp.float32),
        ],
        scratch_types=[
            pltpu.VMEM((bpw,), jnp.int32),
            pltpu.VMEM((bpw,), jnp.int32),
            pltpu.VMEM((bpw // 2, EMBED), jnp.float32),
            pltpu.VMEM((bpw // 2, EMBED), jnp.float32),
            pltpu.SemaphoreType.DMA,
        ],
    )
    def gather_k(uid_hbm, iid_hbm, utab_hbm, itab_hbm, uout_hbm, iout_hbm,
                 uids_v, iids_v, urows, irows, sem):
        wid = lax.axis_index("s") * nc + lax.axis_index("c")
        base = wid * bpw
        half = bpw // 2
        pltpu.sync_copy(uid_hbm.at[wid], uids_v)
        pltpu.sync_copy(iid_hbm.at[wid], iids_v)

        for p in range(2):
            def batch(g, _):
                loc = g * 16
                vu = uids_v[pl.ds(p * half + loc, 16)]
                vi = iids_v[pl.ds(p * half + loc, 16)]
                copies = []
                for l in range(16):
                    copies.append(pltpu.async_copy(
                        utab_hbm.at[_scalar(vu, l)], urows.at[loc + l], sem))
                    copies.append(pltpu.async_copy(
                        itab_hbm.at[_scalar(vi, l)], irows.at[loc + l], sem))
                for c in copies:
                    c.wait()
                return ()

            lax.fori_loop(0, half // 16, batch, (), unroll=False)
            pltpu.sync_copy(urows, uout_hbm.at[pl.ds(base + p * half, half)])
            pltpu.sync_copy(irows, iout_hbm.at[pl.ds(base + p * half, half)])

    return gather_k, nw, bpw


def _mlp_body(u_ref, i_ref, w0u_ref, w0i_ref, b0_ref, w1_ref, b1_ref,
              w2_ref, b2_ref, wo_ref, bo_ref, o_ref):
    h = jnp.dot(u_ref[...], w0u_ref[...], preferred_element_type=jnp.float32)
    h = h + jnp.dot(i_ref[...], w0i_ref[...], preferred_element_type=jnp.float32)
    h = jnp.maximum(h + b0_ref[...], 0.0)
    h = jnp.dot(h, w1_ref[...], preferred_element_type=jnp.float32) + b1_ref[...]
    h = jnp.maximum(h, 0.0)
    h = jnp.dot(h, w2_ref[...], preferred_element_type=jnp.float32) + b2_ref[...]
    h = jnp.maximum(h, 0.0)
    z = jnp.dot(h, wo_ref[...], preferred_element_type=jnp.float32) + bo_ref[...]
    o_ref[...] = 1.0 / (1.0 + jnp.exp(-z))


def _mlp(u, i, W0, b0, W1, b1, W2, b2, Wo, bo, block_m=2048, interpret=False):
    w0u = W0.T[:EMBED]
    w0i = W0.T[EMBED:]
    w1t, w2t, wot = W1.T, W2.T, Wo.T
    b0r, b1r, b2r, bor = b0[None, :], b1[None, :], b2[None, :], bo[None, :]
    grid = (BATCH // block_m,)
    full = lambda m: (0, 0)
    return pl.pallas_call(
        _mlp_body,
        grid=grid,
        in_specs=[
            pl.BlockSpec((block_m, EMBED), lambda m: (m, 0)),
            pl.BlockSpec((block_m, EMBED), lambda m: (m, 0)),
            pl.BlockSpec(w0u.shape, full),
            pl.BlockSpec(w0i.shape, full),
            pl.BlockSpec(b0r.shape, full),
            pl.BlockSpec(w1t.shape, full),
            pl.BlockSpec(b1r.shape, full),
            pl.BlockSpec(w2t.shape, full),
            pl.BlockSpec(b2r.shape, full),
            pl.BlockSpec(wot.shape, full),
            pl.BlockSpec(bor.shape, full),
        ],
        out_specs=pl.BlockSpec((block_m, 1), lambda m: (m, 0)),
        out_shape=jax.ShapeDtypeStruct((BATCH, 1), jnp.float32),
        compiler_params=pltpu.CompilerParams(
            dimension_semantics=("arbitrary",)),
        interpret=interpret,
    )(u, i, w0u, w0i, b0r, w1t, b1r, w2t, b2r, wot, bor)


def _plan(ids, nblocks):
    pos = lax.iota(jnp.int32, BATCH)
    sid, spos = lax.sort([ids, pos], num_keys=1)
    starts = jnp.searchsorted(
        sid, jnp.arange(nblocks, dtype=jnp.int32) * BLK).astype(jnp.int32)
    b = sid // BLK
    rank = pos - starts[b]
    slot = b * SLOTS + rank
    slot_for_batch = jnp.zeros((BATCH,), jnp.int32).at[spos].set(slot)
    idx = starts[:, None] + jnp.arange(SLOTS, dtype=jnp.int32)[None, :]
    ends = jnp.concatenate(
        [starts[1:], jnp.full((1,), BATCH, jnp.int32)])
    valid = idx < ends[:, None]
    blk_ids = jnp.where(valid, sid[jnp.minimum(idx, BATCH - 1)], -1)
    return blk_ids, slot_for_batch


def kernel(user_ids, item_ids, user_table, item_table,
           W0, b0, W1, b1, W2, b2, Wo, bo):
    num_rows = user_table.shape[0]
    nblocks = (num_rows + BLK - 1) // BLK
    ub, uslot = _plan(user_ids.astype(jnp.int32), nblocks)
    ib, islot = _plan(item_ids.astype(jnp.int32), nblocks)
    blk_ids = jnp.concatenate([ub, ib], axis=1)[:, None, :]
    u_slots, i_slots = _extract(user_table.T, item_table.T, blk_ids, nblocks)
    gather_k, nw, bpw = _make_gather(nblocks * SLOTS)
    u_rows, i_rows = gather_k(uslot.reshape(nw, bpw), islot.reshape(nw, bpw),
                              u_slots, i_slots)
    return _mlp(u_rows, i_rows, W0, b0, W1, b1, W2, b2, Wo, bo)


# speedup vs baseline: 1.0547x; 1.0547x over previous
import functools

import jax
import jax.numpy as jnp
from jax import lax
from jax.experimental import pallas as pl
from jax.experimental.pallas import tpu as pltpu
from jax.experimental.pallas import tpu_sc as plsc

BATCH = 16384
EMBED = 64
BLK = 1024
SLOTS = 56


def _scalar(v, l):
    return lax.squeeze(lax.slice(v, (l,), (l + 1,)), (0,))


def _extract_body(nblocks, rem, ids_ref, u_ref, i_ref, uo_ref, io_ref):
    m = pl.program_id(0)
    cols = jax.lax.broadcasted_iota(jnp.int32, (SLOTS, BLK), 1) + m * BLK
    uids = ids_ref[0, 0, :SLOTS][:, None]
    iids = ids_ref[0, 0, SLOTS:][:, None]
    uoh = (cols == uids).astype(jnp.float32)
    ioh = (cols == iids).astype(jnp.float32)
    dn = (((1,), (1,)), ((), ()))

    @pl.when(m != nblocks - 1)
    def _():
        uo_ref[...] = lax.dot_general(uoh, u_ref[...], dn,
                                      preferred_element_type=jnp.float32)
        io_ref[...] = lax.dot_general(ioh, i_ref[...], dn,
                                      preferred_element_type=jnp.float32)

    @pl.when(m == nblocks - 1)
    def _():
        lcol = jax.lax.broadcasted_iota(jnp.int32, (EMBED, BLK), 1)
        xu = jnp.where(lcol < rem, u_ref[...], 0.0)
        xi = jnp.where(lcol < rem, i_ref[...], 0.0)
        uo_ref[...] = lax.dot_general(uoh, xu, dn,
                                      preferred_element_type=jnp.float32)
        io_ref[...] = lax.dot_general(ioh, xi, dn,
                                      preferred_element_type=jnp.float32)


def _extract(tabTu, tabTi, blk_ids, nblocks, interpret=False):
    rem = tabTu.shape[1] - (nblocks - 1) * BLK
    return pl.pallas_call(
        functools.partial(_extract_body, nblocks, rem),
        grid=(nblocks,),
        in_specs=[
            pl.BlockSpec((1, 1, 2 * SLOTS), lambda m: (m, 0, 0)),
            pl.BlockSpec((EMBED, BLK), lambda m: (0, m)),
            pl.BlockSpec((EMBED, BLK), lambda m: (0, m)),
        ],
        out_specs=[
            pl.BlockSpec((SLOTS, EMBED), lambda m: (m, 0)),
            pl.BlockSpec((SLOTS, EMBED), lambda m: (m, 0)),
        ],
        out_shape=[
            jax.ShapeDtypeStruct((nblocks * SLOTS, EMBED), jnp.float32),
            jax.ShapeDtypeStruct((nblocks * SLOTS, EMBED), jnp.float32),
        ],
        compiler_params=pltpu.CompilerParams(
            dimension_semantics=("arbitrary",)),
        interpret=interpret,
    )(blk_ids, tabTu, tabTi)


@functools.lru_cache(maxsize=None)
def _make_gather(num_slots):
    info = plsc.get_sparse_core_info()
    nc, ns = info.num_cores, info.num_subcores
    nw = nc * ns
    bpw = BATCH // nw

    mesh = plsc.VectorSubcoreMesh(core_axis_name="c", subcore_axis_name="s")

    @functools.partial(
        pl.kernel,
        mesh=mesh,
        out_type=[
            jax.ShapeDtypeStruct((BATCH, EMBED), jnp.float32),
            jax.ShapeDtypeStruct((BATCH, EMBED), jnp.float32),
        ],
        scratch_types=[
            pltpu.VMEM((bpw,), jnp.int32),
            pltpu.VMEM((bpw,), jnp.int32),
            pltpu.VMEM((bpw // 2, EMBED), jnp.float32),
            pltpu.VMEM((bpw // 2, EMBED), jnp.float32),
            pltpu.SemaphoreType.DMA,
        ],
    )
    def gather_k(uid_hbm, iid_hbm, utab_hbm, itab_hbm, uout_hbm, iout_hbm,
                 uids_v, iids_v, urows, irows, sem):
        wid = lax.axis_index("s") * nc + lax.axis_index("c")
        base = wid * bpw
        half = bpw // 2
        pltpu.sync_copy(uid_hbm.at[wid], uids_v)
        pltpu.sync_copy(iid_hbm.at[wid], iids_v)

        for p in range(2):
            def batch(g, _):
                loc = g * 16
                vu = uids_v[pl.ds(p * half + loc, 16)]
                vi = iids_v[pl.ds(p * half + loc, 16)]
                copies = []
                for l in range(16):
                    copies.append(pltpu.async_copy(
                        utab_hbm.at[_scalar(vu, l)], urows.at[loc + l], sem))
                    copies.append(pltpu.async_copy(
                        itab_hbm.at[_scalar(vi, l)], irows.at[loc + l], sem))
                for c in copies:
                    c.wait()
                return ()

            lax.fori_loop(0, half // 16, batch, (), unroll=False)
            pltpu.sync_copy(urows, uout_hbm.at[pl.ds(base + p * half, half)])
            pltpu.sync_copy(irows, iout_hbm.at[pl.ds(base + p * half, half)])

    return gather_k, nw, bpw


def _mlp_body(u_ref, i_ref, w0u_ref, w0i_ref, b0_ref, w1_ref, b1_ref,
              w2_ref, b2_ref, wo_ref, bo_ref, o_ref):
    h = jnp.dot(u_ref[...], w0u_ref[...], preferred_element_type=jnp.float32)
    h = h + jnp.dot(i_ref[...], w0i_ref[...], preferred_element_type=jnp.float32)
    h = jnp.maximum(h + b0_ref[...], 0.0)
    h = jnp.dot(h, w1_ref[...], preferred_element_type=jnp.float32) + b1_ref[...]
    h = jnp.maximum(h, 0.0)
    h = jnp.dot(h, w2_ref[...], preferred_element_type=jnp.float32) + b2_ref[...]
    h = jnp.maximum(h, 0.0)
    z = jnp.dot(h, wo_ref[...], preferred_element_type=jnp.float32) + bo_ref[...]
    o_ref[...] = 1.0 / (1.0 + jnp.exp(-z))


def _mlp(u, i, W0, b0, W1, b1, W2, b2, Wo, bo, block_m=2048, interpret=False):
    w0u = W0.T[:EMBED]
    w0i = W0.T[EMBED:]
    w1t, w2t, wot = W1.T, W2.T, Wo.T
    b0r, b1r, b2r, bor = b0[None, :], b1[None, :], b2[None, :], bo[None, :]
    grid = (BATCH // block_m,)
    full = lambda m: (0, 0)
    return pl.pallas_call(
        _mlp_body,
        grid=grid,
        in_specs=[
            pl.BlockSpec((block_m, EMBED), lambda m: (m, 0)),
            pl.BlockSpec((block_m, EMBED), lambda m: (m, 0)),
            pl.BlockSpec(w0u.shape, full),
            pl.BlockSpec(w0i.shape, full),
            pl.BlockSpec(b0r.shape, full),
            pl.BlockSpec(w1t.shape, full),
            pl.BlockSpec(b1r.shape, full),
            pl.BlockSpec(w2t.shape, full),
            pl.BlockSpec(b2r.shape, full),
            pl.BlockSpec(wot.shape, full),
            pl.BlockSpec(bor.shape, full),
        ],
        out_specs=pl.BlockSpec((block_m, 1), lambda m: (m, 0)),
        out_shape=jax.ShapeDtypeStruct((BATCH, 1), jnp.float32),
        compiler_params=pltpu.CompilerParams(
            dimension_semantics=("arbitrary",)),
        interpret=interpret,
    )(u, i, w0u, w0i, b0r, w1t, b1r, w2t, b2r, wot, bor)


def _plan(ids, nblocks):
    pos = lax.iota(jnp.int32, BATCH)
    sid, spos = lax.sort([ids, pos], num_keys=1)
    starts = jnp.searchsorted(
        sid, jnp.arange(nblocks, dtype=jnp.int32) * BLK).astype(jnp.int32)
    b = sid // BLK
    rank = pos - starts[b]
    slot = b * SLOTS + rank
    slot_for_batch = jnp.zeros((BATCH,), jnp.int32).at[spos].set(slot)
    idx = starts[:, None] + jnp.arange(SLOTS, dtype=jnp.int32)[None, :]
    ends = jnp.concatenate(
        [starts[1:], jnp.full((1,), BATCH, jnp.int32)])
    valid = idx < ends[:, None]
    blk_ids = jnp.where(valid, sid[jnp.minimum(idx, BATCH - 1)], -1)
    return blk_ids, slot_for_batch


def kernel(user_ids, item_ids, user_table, item_table,
           W0, b0, W1, b1, W2, b2, Wo, bo):
    num_rows = user_table.shape[0]
    nblocks = (num_rows + BLK - 1) // BLK
    ub, uslot = _plan(user_ids.astype(jnp.int32), nblocks)
    ib, islot = _plan(item_ids.astype(jnp.int32), nblocks)
    blk_ids = jnp.concatenate([ub, ib], axis=1)[:, None, :]
    u_slots, i_slots = _extract(user_table.T, item_table.T, blk_ids, nblocks)
    gather_k, nw, bpw = _make_gather(nblocks * SLOTS)
    u_rows, i_rows = gather_k(uslot.reshape(nw, bpw), islot.reshape(nw, bpw),
                              u_slots, i_slots)
    return _mlp(u_rows, i_rows, W0, b0, W1, b1, W2, b2, Wo, bo)


# speedup vs baseline: 2.6568x; 2.5190x over previous
import functools

import jax
import jax.numpy as jnp
from jax import lax
from jax.experimental import pallas as pl
from jax.experimental.pallas import tpu as pltpu
from jax.experimental.pallas import tpu_sc as plsc

BATCH = 16384
EMBED = 64


@functools.lru_cache(maxsize=None)
def _make_gather(num_users, num_items):
    info = plsc.get_sparse_core_info()
    nc, ns = info.num_cores, info.num_subcores
    nw = nc * ns
    bpw = BATCH // nw

    mesh = plsc.VectorSubcoreMesh(core_axis_name="c", subcore_axis_name="s")

    @functools.partial(
        pl.kernel,
        mesh=mesh,
        out_type=[
            jax.ShapeDtypeStruct((BATCH, EMBED), jnp.float32),
            jax.ShapeDtypeStruct((BATCH, EMBED), jnp.float32),
        ],
        scratch_types=[
            pltpu.VMEM((bpw,), jnp.int32),
            pltpu.VMEM((bpw,), jnp.int32),
            pltpu.VMEM((bpw // 2, EMBED), jnp.float32),
            pltpu.VMEM((bpw // 2, EMBED), jnp.float32),
            pltpu.SemaphoreType.DMA,
        ],
    )
    def gather_k(uid_hbm, iid_hbm, utab_hbm, itab_hbm, uout_hbm, iout_hbm,
                 uids_v, iids_v, urows, irows, sem):
        wid = lax.axis_index("s") * nc + lax.axis_index("c")
        base = wid * bpw
        half = bpw // 2
        pltpu.sync_copy(uid_hbm.at[wid], uids_v)
        pltpu.sync_copy(iid_hbm.at[wid], iids_v)

        def scalar(v, l):
            return lax.squeeze(lax.slice(v, (l,), (l + 1,)), (0,))

        for p in range(2):
            def batch(g, _):
                loc = g * 16
                vu = uids_v[pl.ds(p * half + loc, 16)]
                vi = iids_v[pl.ds(p * half + loc, 16)]
                copies = []
                for l in range(16):
                    copies.append(pltpu.async_copy(
                        utab_hbm.at[scalar(vu, l)], urows.at[loc + l], sem))
                    copies.append(pltpu.async_copy(
                        itab_hbm.at[scalar(vi, l)], irows.at[loc + l], sem))
                for c in copies:
                    c.wait()
                return ()

            lax.fori_loop(0, half // 16, batch, (), unroll=False)
            pltpu.sync_copy(urows, uout_hbm.at[pl.ds(base + p * half, half)])
            pltpu.sync_copy(irows, iout_hbm.at[pl.ds(base + p * half, half)])

    return gather_k, nw, bpw


def _mlp_body(u_ref, i_ref, w0u_ref, w0i_ref, b0_ref, w1_ref, b1_ref,
              w2_ref, b2_ref, wo_ref, bo_ref, o_ref):
    h = jnp.dot(u_ref[...], w0u_ref[...], preferred_element_type=jnp.float32)
    h = h + jnp.dot(i_ref[...], w0i_ref[...], preferred_element_type=jnp.float32)
    h = jnp.maximum(h + b0_ref[...], 0.0)
    h = jnp.dot(h, w1_ref[...], preferred_element_type=jnp.float32) + b1_ref[...]
    h = jnp.maximum(h, 0.0)
    h = jnp.dot(h, w2_ref[...], preferred_element_type=jnp.float32) + b2_ref[...]
    h = jnp.maximum(h, 0.0)
    z = jnp.dot(h, wo_ref[...], preferred_element_type=jnp.float32) + bo_ref[...]
    o_ref[...] = 1.0 / (1.0 + jnp.exp(-z))


def _mlp(u, i, W0, b0, W1, b1, W2, b2, Wo, bo, block_m=2048, interpret=False):
    w0u = W0.T[:EMBED]
    w0i = W0.T[EMBED:]
    w1t, w2t, wot = W1.T, W2.T, Wo.T
    b0r, b1r, b2r, bor = b0[None, :], b1[None, :], b2[None, :], bo[None, :]
    grid = (BATCH // block_m,)
    full = lambda m: (0, 0)
    return pl.pallas_call(
        _mlp_body,
        grid=grid,
        in_specs=[
            pl.BlockSpec((block_m, EMBED), lambda m: (m, 0)),
            pl.BlockSpec((block_m, EMBED), lambda m: (m, 0)),
            pl.BlockSpec(w0u.shape, full),
            pl.BlockSpec(w0i.shape, full),
            pl.BlockSpec(b0r.shape, full),
            pl.BlockSpec(w1t.shape, full),
            pl.BlockSpec(b1r.shape, full),
            pl.BlockSpec(w2t.shape, full),
            pl.BlockSpec(b2r.shape, full),
            pl.BlockSpec(wot.shape, full),
            pl.BlockSpec(bor.shape, full),
        ],
        out_specs=pl.BlockSpec((block_m, 1), lambda m: (m, 0)),
        out_shape=jax.ShapeDtypeStruct((BATCH, 1), jnp.float32),
        compiler_params=pltpu.CompilerParams(
            dimension_semantics=("arbitrary",)),
        interpret=interpret,
    )(u, i, w0u, w0i, b0r, w1t, b1r, w2t, b2r, wot, bor)


def kernel(user_ids, item_ids, user_table, item_table,
           W0, b0, W1, b1, W2, b2, Wo, bo):
    gather_k, nw, bpw = _make_gather(user_table.shape[0], item_table.shape[0])
    uid2 = user_ids.astype(jnp.int32).reshape(nw, bpw)
    iid2 = item_ids.astype(jnp.int32).reshape(nw, bpw)
    u_rows, i_rows = gather_k(uid2, iid2, user_table, item_table)
    return _mlp(u_rows, i_rows, W0, b0, W1, b1, W2, b2, Wo, bo)
